# baseline (speedup 1.0000x reference)
import jax
import jax.numpy as jnp
from jax import lax
from jax.experimental import pallas as pl
from jax.experimental.pallas import tpu as pltpu

_B, _N, _D = 16, 256, 128
_G = 2
_NG = _B // _G
_GR = _G * _N


def _gcn_body(w_ref, b_ref, x_hbm, o_hbm, xbuf, obuf, in_sem, out_sem):
    ii = lax.broadcasted_iota(jnp.int32, (_N, _N), 0)
    jj = lax.broadcasted_iota(jnp.int32, (_N, _N), 1)
    fi = (ii + 1).astype(jnp.float32)
    fj = (jj + 1).astype(jnp.float32)
    m = jnp.where(jj <= ii, lax.rsqrt(fi * fj), 0.0)
    w = w_ref[...]
    b = b_ref[...]
    for g in range(_NG):
        pltpu.make_async_copy(
            x_hbm.at[pl.ds(g * _G, _G)], xbuf.at[g], in_sem.at[g]
        ).start()
    for g in range(_NG):
        pltpu.make_async_copy(
            x_hbm.at[pl.ds(g * _G, _G)], xbuf.at[g], in_sem.at[g]
        ).wait()
        xw = jnp.dot(
            xbuf[g].reshape(_GR, _D), w, preferred_element_type=jnp.float32
        )
        for k in range(_G):
            t = jnp.dot(
                m, xw[k * _N:(k + 1) * _N, :], preferred_element_type=jnp.float32
            )
            obuf[g, pl.ds(k * _N, _N), :] = jnp.maximum(t + b, 0.0)
        pltpu.make_async_copy(
            obuf.at[g], o_hbm.at[pl.ds(g * _GR, _GR)], out_sem.at[g]
        ).start()
    for g in range(_NG):
        pltpu.make_async_copy(
            obuf.at[g], o_hbm.at[pl.ds(g * _GR, _GR)], out_sem.at[g]
        ).wait()


def kernel(x, W, bias):
    bias2 = bias.reshape(1, _D)
    out = pl.pallas_call(
        _gcn_body,
        in_specs=[
            pl.BlockSpec(memory_space=pltpu.MemorySpace.VMEM),
            pl.BlockSpec(memory_space=pltpu.MemorySpace.VMEM),
            pl.BlockSpec(memory_space=pl.ANY),
        ],
        out_specs=pl.BlockSpec(memory_space=pl.ANY),
        out_shape=jax.ShapeDtypeStruct((_B * _N, _D), jnp.float32),
        scratch_shapes=[
            pltpu.VMEM((_NG, _G, _N, _D), jnp.float32),
            pltpu.VMEM((_NG, _GR, _D), jnp.float32),
            pltpu.SemaphoreType.DMA((_NG,)),
            pltpu.SemaphoreType.DMA((_NG,)),
        ],
    )(W, bias2, x)
    return out


# speedup vs baseline: 1.6496x; 1.6496x over previous
import jax
import jax.numpy as jnp
from jax import lax
from jax.experimental import pallas as pl
from jax.experimental.pallas import tpu as pltpu

_B, _N, _D = 16, 256, 128
_BB = 8
_G = 2
_NG = _BB // _G
_GR = _G * _N


def _gcn_body(x_ref, w_ref, b_ref, o_hbm, obuf, sem):
    p = pl.program_id(0)
    ii = lax.broadcasted_iota(jnp.int32, (_N, _N), 0)
    jj = lax.broadcasted_iota(jnp.int32, (_N, _N), 1)
    fi = (ii + 1).astype(jnp.float32)
    fj = (jj + 1).astype(jnp.float32)
    m = jnp.where(jj <= ii, lax.rsqrt(fi * fj), 0.0)
    w = w_ref[...]
    b = b_ref[...]
    xw = jnp.dot(
        x_ref[...].reshape(_BB * _N, _D), w, preferred_element_type=jnp.float32
    )
    base = p * _BB * _N
    for g in range(_NG):
        for k in range(g * _G, (g + 1) * _G):
            t = jnp.dot(
                m, xw[k * _N:(k + 1) * _N, :], preferred_element_type=jnp.float32
            )
            obuf[p, pl.ds(k * _N, _N), :] = jnp.maximum(t + b, 0.0)
        pltpu.make_async_copy(
            obuf.at[p, pl.ds(g * _GR, _GR)],
            o_hbm.at[pl.ds(base + g * _GR, _GR)],
            sem.at[p, g],
        ).start()

    @pl.when(p == _B // _BB - 1)
    def _drain():
        for s in range(_B // _BB):
            for g in range(_NG):
                pltpu.make_async_copy(
                    obuf.at[s, pl.ds(g * _GR, _GR)],
                    o_hbm.at[pl.ds(s * _BB * _N + g * _GR, _GR)],
                    sem.at[s, g],
                ).wait()


def kernel(x, W, bias):
    bias2 = bias.reshape(1, _D)
    out = pl.pallas_call(
        _gcn_body,
        grid=(_B // _BB,),
        in_specs=[
            pl.BlockSpec((_BB, _N, _D), lambda b: (b, 0, 0)),
            pl.BlockSpec((_D, _D), lambda b: (0, 0)),
            pl.BlockSpec((1, _D), lambda b: (0, 0)),
        ],
        out_specs=pl.BlockSpec(memory_space=pl.ANY),
        out_shape=jax.ShapeDtypeStruct((_B * _N, _D), jnp.float32),
        scratch_shapes=[
            pltpu.VMEM((_B // _BB, _BB * _N, _D), jnp.float32),
            pltpu.SemaphoreType.DMA((_B // _BB, _NG)),
        ],
    )(x, W, bias2)
    return out
